# baseline (speedup 1.0000x reference)
import functools

import jax
import jax.numpy as jnp
from jax.experimental import pallas as pl
from jax.experimental.pallas import tpu as pltpu

_BN_EPS = 1e-5
_RING_IN = 4
_RING_OUT = 4


def _mlp2d_kernel(x_hbm, w1_ref, gamma_ref, beta_ref, w2_ref, b2_ref,
                  o_hbm, stage_ref, xs_ref, oring_ref, in_sem, out_sem,
                  *, n_chunks, blk, n_batch, hw):
    cin = w1_ref.shape[1]
    cout = w2_ref.shape[0]
    h_dim = x_hbm.shape[2]
    w_dim = x_hbm.shape[3]

    def in_dma(c):
        return pltpu.make_async_copy(
            x_hbm.at[pl.ds(c * blk, blk)], stage_ref.at[c % _RING_IN],
            in_sem.at[c % _RING_IN])

    def out_dma(j):
        return pltpu.make_async_copy(
            oring_ref.at[j % _RING_OUT], o_hbm.at[j],
            out_sem.at[j % _RING_OUT])

    for c in range(min(_RING_IN, n_chunks)):
        in_dma(c).start()

    colsum = jnp.zeros((cin, 1), jnp.float32)
    gram = jnp.zeros((cin, cin), jnp.float32)
    for c in range(n_chunks):
        in_dma(c).wait()
        for i in range(blk):
            xi = stage_ref[c % _RING_IN, i].reshape(cin, hw)
            colsum += jnp.sum(xi, axis=1, keepdims=True)
            gram += jax.lax.dot_general(
                xi, xi, (((1,), (1,)), ((), ())),
                preferred_element_type=jnp.float32)
            xs_ref[c * blk + i] = xi.astype(jnp.bfloat16)
        if c + _RING_IN < n_chunks:
            in_dma(c + _RING_IN).start()

    sum_h = jax.lax.dot_general(
        w1_ref[...], colsum, (((1,), (0,)), ((), ())),
        preferred_element_type=jnp.float32,
        precision=jax.lax.Precision.HIGHEST)
    wg = jax.lax.dot_general(
        w1_ref[...], gram, (((1,), (0,)), ((), ())),
        preferred_element_type=jnp.float32,
        precision=jax.lax.Precision.HIGHEST)
    sumsq_h = jnp.sum(wg * w1_ref[...], axis=1, keepdims=True)
    inv_count = 1.0 / float(n_batch * hw)
    mean = sum_h * inv_count
    var = jnp.maximum(sumsq_h * inv_count - mean * mean, 0.0)
    scale = gamma_ref[...] * jax.lax.rsqrt(var + _BN_EPS)
    w1s = (scale * w1_ref[...]).astype(jnp.bfloat16)
    shift = beta_ref[...] - mean * scale

    for j in range(n_batch):
        if j >= _RING_OUT:
            out_dma(j - _RING_OUT).wait()
        xi = xs_ref[j]
        h = jnp.dot(w1s, xi, preferred_element_type=jnp.float32)
        h = jnp.maximum(h + shift, 0.0)
        out = jnp.dot(w2_ref[...], h,
                      preferred_element_type=jnp.float32) + b2_ref[...]
        oring_ref[j % _RING_OUT] = (
            out.astype(oring_ref.dtype).reshape(cout, h_dim, w_dim))
        out_dma(j).start()

    for j in range(max(n_batch - _RING_OUT, 0), n_batch):
        out_dma(j).wait()


def kernel(x_nchw, w1, b1, gamma, beta, w2, b2):
    del b1
    N, Cin, H, W = x_nchw.shape
    Cinner = w1.shape[0]
    Cout = w2.shape[0]
    HW = H * W

    n_chunks = next(c for c in (16, 8, 4, 2, 1) if N % c == 0)
    blk = N // n_chunks

    return pl.pallas_call(
        functools.partial(_mlp2d_kernel, n_chunks=n_chunks, blk=blk,
                          n_batch=N, hw=HW),
        in_specs=[
            pl.BlockSpec(memory_space=pl.ANY),
            pl.BlockSpec(memory_space=pltpu.VMEM),
            pl.BlockSpec(memory_space=pltpu.VMEM),
            pl.BlockSpec(memory_space=pltpu.VMEM),
            pl.BlockSpec(memory_space=pltpu.VMEM),
            pl.BlockSpec(memory_space=pltpu.VMEM),
        ],
        out_specs=pl.BlockSpec(memory_space=pl.ANY),
        out_shape=jax.ShapeDtypeStruct((N, Cout, H, W), x_nchw.dtype),
        scratch_shapes=[
            pltpu.VMEM((_RING_IN, blk, Cin, H, W), jnp.float32),
            pltpu.VMEM((N, Cin, HW), jnp.bfloat16),
            pltpu.VMEM((_RING_OUT, Cout, H, W), jnp.float32),
            pltpu.SemaphoreType.DMA((_RING_IN,)),
            pltpu.SemaphoreType.DMA((_RING_OUT,)),
        ],
        compiler_params=pltpu.CompilerParams(
            vmem_limit_bytes=61 * 1024 * 1024,
        ),
        name="mlp2d_fused_native",
    )(x_nchw, w1, gamma, beta, w2, b2)


# speedup vs baseline: 1.0388x; 1.0388x over previous
import functools

import jax
import jax.numpy as jnp
from jax.experimental import pallas as pl
from jax.experimental.pallas import tpu as pltpu

_BN_EPS = 1e-5
_RING_IN = 3
_RING_OUT = 2


def _mlp2d_kernel(x_hbm, w1_ref, gamma_ref, beta_ref, w2_ref, b2_ref,
                  o_hbm, stage_ref, xs_ref, oring_ref, in_sem, out_sem,
                  *, n_chunks, blk, n_batch, hw):
    cin = w1_ref.shape[1]
    cout = w2_ref.shape[0]
    h_dim = x_hbm.shape[2]
    w_dim = x_hbm.shape[3]

    def in_dma(c):
        return pltpu.make_async_copy(
            x_hbm.at[pl.ds(c * blk, blk)], stage_ref.at[c % _RING_IN],
            in_sem.at[c % _RING_IN])

    def out_dma(c):
        return pltpu.make_async_copy(
            oring_ref.at[c % _RING_OUT], o_hbm.at[pl.ds(c * blk, blk)],
            out_sem.at[c % _RING_OUT])

    for c in range(min(_RING_IN, n_chunks)):
        in_dma(c).start()

    colsum = jnp.zeros((cin, 1), jnp.float32)
    gram = jnp.zeros((cin, cin), jnp.float32)
    for c in range(n_chunks):
        in_dma(c).wait()
        for i in range(blk):
            xi = stage_ref[c % _RING_IN, i].reshape(cin, hw)
            colsum += jnp.sum(xi, axis=1, keepdims=True)
            gram += jax.lax.dot_general(
                xi, xi, (((1,), (1,)), ((), ())),
                preferred_element_type=jnp.float32)
            xs_ref[c * blk + i] = xi.astype(jnp.bfloat16)
        if c + _RING_IN < n_chunks:
            in_dma(c + _RING_IN).start()

    sum_h = jax.lax.dot_general(
        w1_ref[...], colsum, (((1,), (0,)), ((), ())),
        preferred_element_type=jnp.float32,
        precision=jax.lax.Precision.HIGHEST)
    wg = jax.lax.dot_general(
        w1_ref[...], gram, (((1,), (0,)), ((), ())),
        preferred_element_type=jnp.float32,
        precision=jax.lax.Precision.HIGHEST)
    sumsq_h = jnp.sum(wg * w1_ref[...], axis=1, keepdims=True)
    inv_count = 1.0 / float(n_batch * hw)
    mean = sum_h * inv_count
    var = jnp.maximum(sumsq_h * inv_count - mean * mean, 0.0)
    scale = gamma_ref[...] * jax.lax.rsqrt(var + _BN_EPS)
    w1s = (scale * w1_ref[...]).astype(jnp.bfloat16)
    shift = beta_ref[...] - mean * scale

    for c in range(n_chunks):
        if c >= _RING_OUT:
            out_dma(c - _RING_OUT).wait()
        for i in range(blk):
            xi = xs_ref[c * blk + i]
            h = jnp.dot(w1s, xi, preferred_element_type=jnp.float32)
            h = jnp.maximum(h + shift, 0.0)
            out = jnp.dot(w2_ref[...], h,
                          preferred_element_type=jnp.float32) + b2_ref[...]
            oring_ref[c % _RING_OUT, i] = (
                out.astype(oring_ref.dtype).reshape(cout, h_dim, w_dim))
        out_dma(c).start()

    for c in range(max(n_chunks - _RING_OUT, 0), n_chunks):
        out_dma(c).wait()


def kernel(x_nchw, w1, b1, gamma, beta, w2, b2):
    del b1
    N, Cin, H, W = x_nchw.shape
    Cinner = w1.shape[0]
    Cout = w2.shape[0]
    HW = H * W

    n_chunks = next(c for c in (16, 8, 4, 2, 1) if N % c == 0)
    blk = N // n_chunks

    return pl.pallas_call(
        functools.partial(_mlp2d_kernel, n_chunks=n_chunks, blk=blk,
                          n_batch=N, hw=HW),
        in_specs=[
            pl.BlockSpec(memory_space=pl.ANY),
            pl.BlockSpec(memory_space=pltpu.VMEM),
            pl.BlockSpec(memory_space=pltpu.VMEM),
            pl.BlockSpec(memory_space=pltpu.VMEM),
            pl.BlockSpec(memory_space=pltpu.VMEM),
            pl.BlockSpec(memory_space=pltpu.VMEM),
        ],
        out_specs=pl.BlockSpec(memory_space=pl.ANY),
        out_shape=jax.ShapeDtypeStruct((N, Cout, H, W), x_nchw.dtype),
        scratch_shapes=[
            pltpu.VMEM((_RING_IN, blk, Cin, H, W), jnp.float32),
            pltpu.VMEM((N, Cin, HW), jnp.bfloat16),
            pltpu.VMEM((_RING_OUT, blk, Cout, H, W), jnp.float32),
            pltpu.SemaphoreType.DMA((_RING_IN,)),
            pltpu.SemaphoreType.DMA((_RING_OUT,)),
        ],
        compiler_params=pltpu.CompilerParams(
            vmem_limit_bytes=61 * 1024 * 1024,
        ),
        name="mlp2d_fused_native",
    )(x_nchw, w1, gamma, beta, w2, b2)


# speedup vs baseline: 1.0504x; 1.0112x over previous
import functools

import jax
import jax.numpy as jnp
from jax.experimental import pallas as pl
from jax.experimental.pallas import tpu as pltpu

_BN_EPS = 1e-5
_RING_IN = 3
_RING_OUT = 2


def _mlp2d_kernel(x_hbm, w1_ref, gamma_ref, beta_ref, w2_ref, b2_ref,
                  o_hbm, stage_ref, xs_ref, oring_ref, in_sem, out_sem,
                  *, n_chunks, blk, n_batch, hw):
    cin = w1_ref.shape[1]
    cout = w2_ref.shape[0]
    h_dim = x_hbm.shape[2]
    w_dim = x_hbm.shape[3]

    def in_dma(c):
        return pltpu.make_async_copy(
            x_hbm.at[pl.ds(c * blk, blk)], stage_ref.at[c % _RING_IN],
            in_sem.at[c % _RING_IN])

    def out_dma(c):
        return pltpu.make_async_copy(
            oring_ref.at[c % _RING_OUT], o_hbm.at[pl.ds(c * blk, blk)],
            out_sem.at[c % _RING_OUT])

    for c in range(min(_RING_IN, n_chunks)):
        in_dma(c).start()

    colsum = jnp.zeros((cin, 1), jnp.float32)
    gram = jnp.zeros((cin, cin), jnp.float32)
    for c in range(n_chunks):
        in_dma(c).wait()
        for i in range(blk):
            x4 = stage_ref[c % _RING_IN, i]
            xi = x4.astype(jnp.bfloat16).reshape(cin, hw)
            colsum += jnp.sum(xi.astype(jnp.float32), axis=1, keepdims=True)
            gram += jax.lax.dot_general(
                xi, xi, (((1,), (1,)), ((), ())),
                preferred_element_type=jnp.float32)
            xs_ref[c * blk + i] = xi
        if c + _RING_IN < n_chunks:
            in_dma(c + _RING_IN).start()

    sum_h = jax.lax.dot_general(
        w1_ref[...], colsum, (((1,), (0,)), ((), ())),
        preferred_element_type=jnp.float32,
        precision=jax.lax.Precision.HIGHEST)
    wg = jax.lax.dot_general(
        w1_ref[...], gram, (((1,), (0,)), ((), ())),
        preferred_element_type=jnp.float32,
        precision=jax.lax.Precision.HIGHEST)
    sumsq_h = jnp.sum(wg * w1_ref[...], axis=1, keepdims=True)
    inv_count = 1.0 / float(n_batch * hw)
    mean = sum_h * inv_count
    var = jnp.maximum(sumsq_h * inv_count - mean * mean, 0.0)
    scale = gamma_ref[...] * jax.lax.rsqrt(var + _BN_EPS)
    w1s = (scale * w1_ref[...]).astype(jnp.bfloat16)
    shift = beta_ref[...] - mean * scale

    for c in range(n_chunks):
        if c >= _RING_OUT:
            out_dma(c - _RING_OUT).wait()
        for i in range(blk):
            xi = xs_ref[c * blk + i]
            h = jnp.dot(w1s, xi, preferred_element_type=jnp.float32)
            h = jnp.maximum(h + shift, 0.0)
            out = jnp.dot(w2_ref[...], h,
                          preferred_element_type=jnp.float32) + b2_ref[...]
            oring_ref[c % _RING_OUT, i] = (
                out.astype(oring_ref.dtype).reshape(cout, h_dim, w_dim))
        out_dma(c).start()

    for c in range(max(n_chunks - _RING_OUT, 0), n_chunks):
        out_dma(c).wait()


def kernel(x_nchw, w1, b1, gamma, beta, w2, b2):
    del b1
    N, Cin, H, W = x_nchw.shape
    Cinner = w1.shape[0]
    Cout = w2.shape[0]
    HW = H * W

    n_chunks = next(c for c in (16, 8, 4, 2, 1) if N % c == 0)
    blk = N // n_chunks

    return pl.pallas_call(
        functools.partial(_mlp2d_kernel, n_chunks=n_chunks, blk=blk,
                          n_batch=N, hw=HW),
        in_specs=[
            pl.BlockSpec(memory_space=pl.ANY),
            pl.BlockSpec(memory_space=pltpu.VMEM),
            pl.BlockSpec(memory_space=pltpu.VMEM),
            pl.BlockSpec(memory_space=pltpu.VMEM),
            pl.BlockSpec(memory_space=pltpu.VMEM),
            pl.BlockSpec(memory_space=pltpu.VMEM),
        ],
        out_specs=pl.BlockSpec(memory_space=pl.ANY),
        out_shape=jax.ShapeDtypeStruct((N, Cout, H, W), x_nchw.dtype),
        scratch_shapes=[
            pltpu.VMEM((_RING_IN, blk, Cin, H, W), jnp.float32),
            pltpu.VMEM((N, Cin, HW), jnp.bfloat16),
            pltpu.VMEM((_RING_OUT, blk, Cout, H, W), jnp.float32),
            pltpu.SemaphoreType.DMA((_RING_IN,)),
            pltpu.SemaphoreType.DMA((_RING_OUT,)),
        ],
        compiler_params=pltpu.CompilerParams(
            vmem_limit_bytes=61 * 1024 * 1024,
        ),
        name="mlp2d_fused_native",
    )(x_nchw, w1, gamma, beta, w2, b2)
